# Initial kernel scaffold; baseline (speedup 1.0000x reference)
import jax
import jax.numpy as jnp
from jax.experimental import pallas as pl


def kernel(dist, angle, idx_t, index_t, index_h):
    raise NotImplementedError("write your pallas kernel here")


# speedup vs baseline: 10.2211x; 10.2211x over previous
import functools

import jax
import jax.numpy as jnp
from jax import lax
from jax.experimental import pallas as pl
from jax.experimental.pallas import tpu as pltpu
from jax.experimental.pallas import tpu_sc as plsc

_B, _L, _A = 4096, 64, 4
_H, _W = 50000, 32
_LP = _L + 1
_V = _B * _LP
_NC, _NS = 2, 16
_NW = _NC * _NS
_CHUNK = 1568
_RB = 16
_D1 = 704
_D2 = 736
_DOUT = 992


_TBLK = 64


def _tables_body(a_ref, d_ref, ta_ref, td_ref):
    a = a_ref[...]
    ta_ref[:, 0:_L, 0:_A] = jnp.sin(a)
    ta_ref[:, 0:_L, _A:2 * _A] = jnp.cos(a)
    ta_ref[:, _L:_LP, :] = jnp.zeros((_TBLK, 1, 2 * _A), jnp.float32)
    td_ref[...] = jnp.zeros((_TBLK, _LP, 8), jnp.float32)
    td_ref[:, 0:_L, 0:1] = d_ref[...][:, :, None]


def _build_tables(dist, angle):
    grid = _B // _TBLK
    return pl.pallas_call(
        _tables_body,
        grid=(grid,),
        in_specs=[
            pl.BlockSpec((_TBLK, _L, _A), lambda i: (i, 0, 0)),
            pl.BlockSpec((_TBLK, _L), lambda i: (i, 0)),
        ],
        out_specs=[
            pl.BlockSpec((_TBLK, _LP, 2 * _A), lambda i: (i, 0, 0)),
            pl.BlockSpec((_TBLK, _LP, 8), lambda i: (i, 0, 0)),
        ],
        out_shape=[
            jax.ShapeDtypeStruct((_B, _LP, 2 * _A), jnp.float32),
            jax.ShapeDtypeStruct((_B, _LP, 8), jnp.float32),
        ],
    )(angle, dist)


@functools.lru_cache(maxsize=None)
def _mesh():
    return plsc.VectorSubcoreMesh(
        core_axis_name="c", subcore_axis_name="s",
        num_cores=_NC, num_subcores=_NS,
    )


def _worker_id():
    return lax.axis_index("s") * _NC + lax.axis_index("c")


def _k1_body(ih_hbm, it_hbm, td_hbm, fi_out, dv_out, part_out,
             ihB, itB, fiB, dbB, dvB, stage, sem):
    wid = _worker_id()
    r_start = wid * _CHUNK
    nrows = jnp.minimum(_CHUNK, _H - r_start)
    nblk = nrows // _RB
    iota16 = lax.iota(jnp.int32, 16)
    zcol = jnp.zeros((16,), jnp.int32)

    def blk(b, carry):
        mn, mx = carry
        r0 = r_start + b * _RB
        pltpu.sync_copy(ih_hbm.at[pl.ds(r0, _RB)], ihB)
        pltpu.sync_copy(it_hbm.at[pl.ds(r0, _RB)], itB)
        ihv = ihB[...] * _LP
        for h in range(_RB):
            base = ihv[h]
            for k in range(2):
                v = itB[h, pl.ds(16 * k, 16)] + base
                flat = 32 * h + 16 * k
                fiB[flat // 128, pl.ds(flat % 128, 16)] = v
        descs = [
            pltpu.async_copy(td_hbm.at[fiB.at[g]],
                             dbB.at[pl.ds(128 * g, 128)], sem)
            for g in range(4)
        ]
        for dsc in descs:
            dsc.wait()
        for h in range(_RB):
            for k in range(2):
                flat = 32 * h + 16 * k
                vals = plsc.load_gather(dbB, [flat + iota16, zcol])
                dvB[pl.ds(flat, 16)] = vals
                mn = jnp.minimum(mn, vals)
                mx = jnp.maximum(mx, vals)
        for g in range(4):
            pltpu.sync_copy(fiB.at[g],
                            fi_out.at[pl.ds(r0 * _W + 128 * g, 128)])
        pltpu.sync_copy(dvB, dv_out.at[pl.ds(r0 * _W, _RB * _W)])
        return mn, mx

    big = jnp.full((16,), 3.0e38, jnp.float32)
    mn, mx = lax.fori_loop(0, nblk, blk, (big, -big))
    stage[pl.ds(0, 16)] = mn
    stage[pl.ds(16, 16)] = mx
    pltpu.sync_copy(stage, part_out.at[pl.ds(32 * wid, 32)])


@functools.lru_cache(maxsize=None)
def _k1():
    return pl.kernel(
        _k1_body,
        out_type=(
            jax.ShapeDtypeStruct((_H * _W,), jnp.int32),
            jax.ShapeDtypeStruct((_H * _W,), jnp.float32),
            jax.ShapeDtypeStruct((_NW * 32,), jnp.float32),
        ),
        mesh=_mesh(),
        compiler_params=pltpu.CompilerParams(needs_layout_passes=False, use_tc_tiling_on_sc=False),
        scratch_types=[
            pltpu.VMEM((_RB,), jnp.int32),
            pltpu.VMEM((_RB, _W), jnp.int32),
            pltpu.VMEM((4, 128), jnp.int32),
            pltpu.VMEM((_RB * _W, 8), jnp.float32),
            pltpu.VMEM((_RB * _W,), jnp.float32),
            pltpu.VMEM((32,), jnp.float32),
            pltpu.SemaphoreType.DMA,
        ],
    )


def _k2_body(fi_hbm, dv_hbm, idt_hbm, ta_hbm, part_hbm, out_hbm,
             fiB, dvB, idB, xd, pbuf, sem):
    wid = _worker_id()
    r_start = wid * _CHUNK
    nrows = jnp.minimum(_CHUNK, _H - r_start)
    nblk = nrows // _RB
    iota16 = lax.iota(jnp.int32, 16)
    ones16 = jnp.ones((16,), jnp.float32)
    zero16 = jnp.zeros((16,), jnp.float32)
    sub8 = lax.shift_right_logical(iota16, 3)
    lane8 = lax.bitwise_and(iota16, 7)

    pltpu.sync_copy(part_hbm, pbuf)
    mn = pbuf[pl.ds(0, 16)]
    mx = pbuf[pl.ds(16, 16)]
    for i in range(1, _NW):
        mn = jnp.minimum(mn, pbuf[pl.ds(32 * i, 16)])
        mx = jnp.maximum(mx, pbuf[pl.ds(32 * i + 16, 16)])
    gmin = jnp.min(mn)
    inv_v = ones16 / (jnp.full((16,), 1.0, jnp.float32) * (jnp.max(mx) - gmin))

    pos_base = [22 * (16 * k + iota16) for k in range(2)]

    def blk(b, _):
        r0 = r_start + b * _RB
        for g in range(4):
            pltpu.sync_copy(fi_hbm.at[pl.ds(r0 * _W + 128 * g, 128)],
                            fiB.at[g])
        pltpu.sync_copy(dv_hbm.at[pl.ds(r0 * _W, _RB * _W)], dvB)
        pltpu.sync_copy(idt_hbm.at[pl.ds(r0, _RB)], idB)
        descs = [
            pltpu.async_copy(
                ta_hbm.at[fiB.at[h // 4, pl.ds(32 * (h % 4), 32)]],
                xd.at[h, pl.ds(_D2 // 8, _W), :], sem)
            for h in range(_RB)
        ]
        for h in range(_RB):
            hsplat = jnp.full((16,), h, jnp.int32)
            for j in range(_D1 // 16):
                plsc.store_scatter(xd, [hsplat, 2 * j + sub8, lane8], zero16)
            for k in range(2):
                c = idB[h, pl.ds(16 * k, 16)]
                flat = pos_base[k] + c
                plsc.store_scatter(
                    xd,
                    [hsplat, lax.shift_right_logical(flat, 3),
                     lax.bitwise_and(flat, 7)],
                    ones16)
                d = dvB[pl.ds(32 * h + 16 * k, 16)]
                f2 = _D1 + 16 * k + iota16
                plsc.store_scatter(
                    xd,
                    [hsplat, lax.shift_right_logical(f2, 3),
                     lax.bitwise_and(f2, 7)],
                    (d - gmin) * inv_v)
        for dsc in descs:
            dsc.wait()
        pltpu.sync_copy(xd, out_hbm.at[pl.ds(r0, _RB)])
        return 0

    lax.fori_loop(0, nblk, blk, 0)


@functools.lru_cache(maxsize=None)
def _k2():
    return pl.kernel(
        _k2_body,
        out_type=jax.ShapeDtypeStruct((_H, _DOUT // 8, 8), jnp.float32),
        mesh=_mesh(),
        compiler_params=pltpu.CompilerParams(needs_layout_passes=False, use_tc_tiling_on_sc=False),
        scratch_types=[
            pltpu.VMEM((4, 128), jnp.int32),
            pltpu.VMEM((_RB * _W,), jnp.float32),
            pltpu.VMEM((_RB, _W), jnp.int32),
            pltpu.VMEM((_RB, _DOUT // 8, 8), jnp.float32),
            pltpu.VMEM((_NW * 32,), jnp.float32),
            pltpu.SemaphoreType.DMA,
        ],
    )


def kernel(dist, angle, idx_t, index_t, index_h):
    ta3, td3 = _build_tables(dist, angle)
    ta = ta3.reshape(_V, 2 * _A)
    td = td3.reshape(_V, 8)
    fi, dv, part = _k1()(index_h, index_t, td)
    out3 = _k2()(fi, dv, idx_t, ta, part)
    return out3.reshape(_H, _DOUT)


# speedup vs baseline: 18.2888x; 1.7893x over previous
import functools

import jax
import jax.numpy as jnp
from jax import lax
from jax.experimental import pallas as pl
from jax.experimental.pallas import tpu as pltpu
from jax.experimental.pallas import tpu_sc as plsc

_B, _L, _A = 4096, 64, 4
_H, _W = 50000, 32
_LP = _L + 1
_V = _B * _LP
_NC, _NS = 2, 16
_NW = _NC * _NS
_CHUNK = 1568
_RB = 16
_D1 = 704
_D2 = 736
_DOUT = 992


_TBLK = 64


def _tables_body(a_ref, d_ref, ta_ref, td_ref):
    a = a_ref[...]
    ta_ref[:, 0:_L, 0:_A] = jnp.sin(a)
    ta_ref[:, 0:_L, _A:2 * _A] = jnp.cos(a)
    ta_ref[:, _L:_LP, :] = jnp.zeros((_TBLK, 1, 2 * _A), jnp.float32)
    td_ref[...] = jnp.zeros((_TBLK, _LP, 8), jnp.float32)
    td_ref[:, 0:_L, 0:1] = d_ref[...][:, :, None]


def _build_tables(dist, angle):
    grid = _B // _TBLK
    return pl.pallas_call(
        _tables_body,
        grid=(grid,),
        in_specs=[
            pl.BlockSpec((_TBLK, _L, _A), lambda i: (i, 0, 0)),
            pl.BlockSpec((_TBLK, _L), lambda i: (i, 0)),
        ],
        out_specs=[
            pl.BlockSpec((_TBLK, _LP, 2 * _A), lambda i: (i, 0, 0)),
            pl.BlockSpec((_TBLK, _LP, 8), lambda i: (i, 0, 0)),
        ],
        out_shape=[
            jax.ShapeDtypeStruct((_B, _LP, 2 * _A), jnp.float32),
            jax.ShapeDtypeStruct((_B, _LP, 8), jnp.float32),
        ],
    )(angle, dist)


@functools.lru_cache(maxsize=None)
def _mesh():
    return plsc.VectorSubcoreMesh(
        core_axis_name="c", subcore_axis_name="s",
        num_cores=_NC, num_subcores=_NS,
    )


def _worker_id():
    return lax.axis_index("s") * _NC + lax.axis_index("c")


def _k1_body(ih_hbm, it_hbm, td_hbm, fi_out, dv_out, part_out,
             ihB, itB, fiB, dbB, dvB, stage, sem):
    wid = _worker_id()
    r_start = wid * _CHUNK
    nrows = jnp.minimum(_CHUNK, _H - r_start)
    nblk = nrows // _RB
    iota16 = lax.iota(jnp.int32, 16)
    zcol = jnp.zeros((16,), jnp.int32)

    def blk(b, carry):
        mn, mx = carry
        r0 = r_start + b * _RB
        pltpu.sync_copy(ih_hbm.at[pl.ds(r0, _RB)], ihB)
        pltpu.sync_copy(it_hbm.at[pl.ds(r0, _RB)], itB)
        ihv = ihB[...] * _LP
        for h in range(_RB):
            base = ihv[h]
            for k in range(2):
                v = itB[h, pl.ds(16 * k, 16)] + base
                flat = 32 * h + 16 * k
                fiB[flat // 128, pl.ds(flat % 128, 16)] = v
        descs = [
            pltpu.async_copy(td_hbm.at[fiB.at[g]],
                             dbB.at[pl.ds(128 * g, 128)], sem)
            for g in range(4)
        ]
        for dsc in descs:
            dsc.wait()
        for h in range(_RB):
            for k in range(2):
                flat = 32 * h + 16 * k
                vals = plsc.load_gather(dbB, [flat + iota16, zcol])
                dvB[pl.ds(flat, 16)] = vals
                mn = jnp.minimum(mn, vals)
                mx = jnp.maximum(mx, vals)
        for g in range(4):
            pltpu.sync_copy(fiB.at[g],
                            fi_out.at[pl.ds(r0 * _W + 128 * g, 128)])
        pltpu.sync_copy(dvB, dv_out.at[pl.ds(r0 * _W, _RB * _W)])
        return mn, mx

    big = jnp.full((16,), 3.0e38, jnp.float32)
    mn, mx = lax.fori_loop(0, nblk, blk, (big, -big))
    stage[pl.ds(0, 16)] = mn
    stage[pl.ds(16, 16)] = mx
    pltpu.sync_copy(stage, part_out.at[pl.ds(32 * wid, 32)])


@functools.lru_cache(maxsize=None)
def _k1():
    return pl.kernel(
        _k1_body,
        out_type=(
            jax.ShapeDtypeStruct((_H * _W,), jnp.int32),
            jax.ShapeDtypeStruct((_H * _W,), jnp.float32),
            jax.ShapeDtypeStruct((_NW * 32,), jnp.float32),
        ),
        mesh=_mesh(),
        compiler_params=pltpu.CompilerParams(needs_layout_passes=False, use_tc_tiling_on_sc=False),
        scratch_types=[
            pltpu.VMEM((_RB,), jnp.int32),
            pltpu.VMEM((_RB, _W), jnp.int32),
            pltpu.VMEM((4, 128), jnp.int32),
            pltpu.VMEM((_RB * _W, 8), jnp.float32),
            pltpu.VMEM((_RB * _W,), jnp.float32),
            pltpu.VMEM((32,), jnp.float32),
            pltpu.SemaphoreType.DMA,
        ],
    )


def _k2_body(fi_hbm, dv_hbm, idt_hbm, ta_hbm, part_hbm, out_hbm,
             fiB, dvB, idB, abB, xd, pbuf, sem):
    wid = _worker_id()
    r_start = wid * _CHUNK
    nrows = jnp.minimum(_CHUNK, _H - r_start)
    nblk = nrows // _RB
    iota16 = lax.iota(jnp.int32, 16)
    ones16 = jnp.ones((16,), jnp.float32)
    zero16 = jnp.zeros((16,), jnp.float32)
    rbase = lax.shift_right_logical(iota16, 3)
    cidx = lax.bitwise_and(iota16, 7)

    pltpu.sync_copy(part_hbm, pbuf)
    mn = pbuf[pl.ds(0, 16)]
    mx = pbuf[pl.ds(16, 16)]
    for i in range(1, _NW):
        mn = jnp.minimum(mn, pbuf[pl.ds(32 * i, 16)])
        mx = jnp.maximum(mx, pbuf[pl.ds(32 * i + 16, 16)])
    gmin = jnp.min(mn)
    inv_v = ones16 / (jnp.full((16,), 1.0, jnp.float32) * (jnp.max(mx) - gmin))

    pos_base = [22 * (16 * k + iota16) for k in range(2)]

    def blk(b, _):
        r0 = r_start + b * _RB
        for g in range(4):
            pltpu.sync_copy(fi_hbm.at[pl.ds(r0 * _W + 128 * g, 128)],
                            fiB.at[g])
        pltpu.sync_copy(dv_hbm.at[pl.ds(r0 * _W, _RB * _W)], dvB)
        pltpu.sync_copy(idt_hbm.at[pl.ds(r0, _RB)], idB)
        descs = [
            pltpu.async_copy(ta_hbm.at[fiB.at[g]],
                             abB.at[pl.ds(128 * g, 128)], sem)
            for g in range(4)
        ]
        for h in range(_RB):
            hsplat = jnp.full((16,), h, jnp.int32)
            for j in range(_D1 // 16):
                xd[h, pl.ds(16 * j, 16)] = zero16
            for k in range(2):
                c = idB[h, pl.ds(16 * k, 16)]
                plsc.store_scatter(xd, [hsplat, pos_base[k] + c], ones16)
                d = dvB[pl.ds(32 * h + 16 * k, 16)]
                xd[h, pl.ds(_D1 + 16 * k, 16)] = (d - gmin) * inv_v
        for dsc in descs:
            dsc.wait()
        for h in range(_RB):
            for j in range(16):
                v = plsc.load_gather(abB, [32 * h + 2 * j + rbase, cidx])
                xd[h, pl.ds(_D2 + 16 * j, 16)] = v
        pltpu.sync_copy(xd, out_hbm.at[pl.ds(r0, _RB)])
        return 0

    lax.fori_loop(0, nblk, blk, 0)


@functools.lru_cache(maxsize=None)
def _k2():
    return pl.kernel(
        _k2_body,
        out_type=jax.ShapeDtypeStruct((_H, _DOUT), jnp.float32),
        mesh=_mesh(),
        compiler_params=pltpu.CompilerParams(needs_layout_passes=False, use_tc_tiling_on_sc=False),
        scratch_types=[
            pltpu.VMEM((4, 128), jnp.int32),
            pltpu.VMEM((_RB * _W,), jnp.float32),
            pltpu.VMEM((_RB, _W), jnp.int32),
            pltpu.VMEM((_RB * _W, 8), jnp.float32),
            pltpu.VMEM((_RB, _DOUT), jnp.float32),
            pltpu.VMEM((_NW * 32,), jnp.float32),
            pltpu.SemaphoreType.DMA,
        ],
    )


def kernel(dist, angle, idx_t, index_t, index_h):
    ta3, td3 = _build_tables(dist, angle)
    ta = ta3.reshape(_V, 2 * _A)
    td = td3.reshape(_V, 8)
    fi, dv, part = _k1()(index_h, index_t, td)
    return _k2()(fi, dv, idx_t, ta, part)


# speedup vs baseline: 24.4268x; 1.3356x over previous
import functools

import jax
import jax.numpy as jnp
from jax import lax
from jax.experimental import pallas as pl
from jax.experimental.pallas import tpu as pltpu
from jax.experimental.pallas import tpu_sc as plsc

_B, _L, _A = 4096, 64, 4
_H, _W = 50000, 32
_LP = _L + 1
_V = _B * _LP
_NC, _NS = 2, 16
_NW = _NC * _NS
_CHUNK = 1568
_RB = 16
_D1 = 704
_D2 = 736
_DOUT = 992


_TBLK = 64


def _tables_body(a_ref, d_ref, ta_ref, td_ref):
    a = a_ref[...]
    ta_ref[:, 0:_L, 0:_A] = jnp.sin(a)
    ta_ref[:, 0:_L, _A:2 * _A] = jnp.cos(a)
    ta_ref[:, _L:_LP, :] = jnp.zeros((_TBLK, 1, 2 * _A), jnp.float32)
    td_ref[...] = jnp.zeros((_TBLK, _LP, 8), jnp.float32)
    td_ref[:, 0:_L, 0:1] = d_ref[...][:, :, None]


def _build_tables(dist, angle):
    grid = _B // _TBLK
    return pl.pallas_call(
        _tables_body,
        grid=(grid,),
        in_specs=[
            pl.BlockSpec((_TBLK, _L, _A), lambda i: (i, 0, 0)),
            pl.BlockSpec((_TBLK, _L), lambda i: (i, 0)),
        ],
        out_specs=[
            pl.BlockSpec((_TBLK, _LP, 2 * _A), lambda i: (i, 0, 0)),
            pl.BlockSpec((_TBLK, _LP, 8), lambda i: (i, 0, 0)),
        ],
        out_shape=[
            jax.ShapeDtypeStruct((_B, _LP, 2 * _A), jnp.float32),
            jax.ShapeDtypeStruct((_B, _LP, 8), jnp.float32),
        ],
    )(angle, dist)


@functools.lru_cache(maxsize=None)
def _mesh():
    return plsc.VectorSubcoreMesh(
        core_axis_name="c", subcore_axis_name="s",
        num_cores=_NC, num_subcores=_NS,
    )


def _worker_id():
    return lax.axis_index("s") * _NC + lax.axis_index("c")


def _k1_body(ih_hbm, itT_hbm, td_hbm, fi_out, dv_out, part_out,
             ihB, itB, fiB, dbB, dvB, stage, sem):
    wid = _worker_id()
    r_start = wid * _CHUNK
    nrows = jnp.minimum(_CHUNK, _H - r_start)
    nblk = nrows // _RB
    iota16 = lax.iota(jnp.int32, 16)
    zcol = jnp.zeros((16,), jnp.int32)

    def blk(b, carry):
        mn, mx = carry
        r0 = r_start + b * _RB
        pltpu.sync_copy(ih_hbm.at[pl.ds(r0, _RB)], ihB)
        pltpu.sync_copy(itT_hbm.at[:, pl.ds(r0, _RB)], itB)
        ihv = ihB[...] * _LP
        for w in range(_W):
            v = itB[w, :] + ihv
            fiB[w // 8, pl.ds(16 * (w % 8), 16)] = v
        descs = [
            pltpu.async_copy(td_hbm.at[fiB.at[g]],
                             dbB.at[pl.ds(128 * g, 128)], sem)
            for g in range(4)
        ]
        for dsc in descs:
            dsc.wait()
        for w in range(_W):
            vals = plsc.load_gather(dbB, [16 * w + iota16, zcol])
            dvB[pl.ds(16 * w, 16)] = vals
            mn = jnp.minimum(mn, vals)
            mx = jnp.maximum(mx, vals)
        for g in range(4):
            pltpu.sync_copy(fiB.at[g],
                            fi_out.at[pl.ds(r0 * _W + 128 * g, 128)])
        pltpu.sync_copy(dvB, dv_out.at[pl.ds(r0 * _W, _RB * _W)])
        return mn, mx

    big = jnp.full((16,), 3.0e38, jnp.float32)
    mn, mx = lax.fori_loop(0, nblk, blk, (big, -big))
    stage[pl.ds(0, 16)] = mn
    stage[pl.ds(16, 16)] = mx
    pltpu.sync_copy(stage, part_out.at[pl.ds(32 * wid, 32)])


@functools.lru_cache(maxsize=None)
def _k1():
    return pl.kernel(
        _k1_body,
        out_type=(
            jax.ShapeDtypeStruct((_H * _W,), jnp.int32),
            jax.ShapeDtypeStruct((_H * _W,), jnp.float32),
            jax.ShapeDtypeStruct((_NW * 32,), jnp.float32),
        ),
        mesh=_mesh(),
        compiler_params=pltpu.CompilerParams(needs_layout_passes=False, use_tc_tiling_on_sc=False),
        scratch_types=[
            pltpu.VMEM((_RB,), jnp.int32),
            pltpu.VMEM((_W, _RB), jnp.int32),
            pltpu.VMEM((4, 128), jnp.int32),
            pltpu.VMEM((_RB * _W, 8), jnp.float32),
            pltpu.VMEM((_RB * _W,), jnp.float32),
            pltpu.VMEM((32,), jnp.float32),
            pltpu.SemaphoreType.DMA,
        ],
    )


def _k2_body(fi_hbm, dv_hbm, idtT_hbm, ta_hbm, part_hbm, outT_hbm,
             fiB, dvB, idB, abB, xd, pbuf, sem):
    wid = _worker_id()
    r_start = wid * _CHUNK
    nrows = jnp.minimum(_CHUNK, _H - r_start)
    nblk = nrows // _RB
    iota16 = lax.iota(jnp.int32, 16)
    ones16 = jnp.ones((16,), jnp.float32)
    zero16 = jnp.zeros((16,), jnp.float32)

    pltpu.sync_copy(part_hbm, pbuf)
    mn = pbuf[pl.ds(0, 16)]
    mx = pbuf[pl.ds(16, 16)]
    for i in range(1, _NW):
        mn = jnp.minimum(mn, pbuf[pl.ds(32 * i, 16)])
        mx = jnp.maximum(mx, pbuf[pl.ds(32 * i + 16, 16)])
    gmin = jnp.min(mn)
    inv_v = ones16 / (jnp.full((16,), 1.0, jnp.float32) * (jnp.max(mx) - gmin))

    def blk(b, _):
        r0 = r_start + b * _RB
        for g in range(4):
            pltpu.sync_copy(fi_hbm.at[pl.ds(r0 * _W + 128 * g, 128)],
                            fiB.at[g])
        pltpu.sync_copy(dv_hbm.at[pl.ds(r0 * _W, _RB * _W)], dvB)
        pltpu.sync_copy(idtT_hbm.at[:, pl.ds(r0, _RB)], idB)
        descs = [
            pltpu.async_copy(ta_hbm.at[fiB.at[g]],
                             abB.at[pl.ds(128 * g, 128)], sem)
            for g in range(4)
        ]
        for c in range(_D1):
            xd[c, :] = zero16
        for w in range(_W):
            cvals = idB[w, :]
            plsc.store_scatter(xd, [22 * w + cvals, iota16], ones16)
            d = dvB[pl.ds(16 * w, 16)]
            xd[_D1 + w, :] = (d - gmin) * inv_v
        for dsc in descs:
            dsc.wait()
        for w in range(_W):
            base = 16 * w + iota16
            for j in range(8):
                v = plsc.load_gather(abB, [base, jnp.full((16,), j, jnp.int32)])
                xd[_D2 + 8 * w + j, :] = v
        pltpu.sync_copy(xd, outT_hbm.at[:, pl.ds(r0, _RB)])
        return 0

    lax.fori_loop(0, nblk, blk, 0)


@functools.lru_cache(maxsize=None)
def _k2():
    return pl.kernel(
        _k2_body,
        out_type=jax.ShapeDtypeStruct((_DOUT, _H), jnp.float32),
        mesh=_mesh(),
        compiler_params=pltpu.CompilerParams(needs_layout_passes=False, use_tc_tiling_on_sc=False),
        scratch_types=[
            pltpu.VMEM((4, 128), jnp.int32),
            pltpu.VMEM((_RB * _W,), jnp.float32),
            pltpu.VMEM((_W, _RB), jnp.int32),
            pltpu.VMEM((_RB * _W, 8), jnp.float32),
            pltpu.VMEM((_DOUT, _RB), jnp.float32),
            pltpu.VMEM((_NW * 32,), jnp.float32),
            pltpu.SemaphoreType.DMA,
        ],
    )


def kernel(dist, angle, idx_t, index_t, index_h):
    ta3, td3 = _build_tables(dist, angle)
    ta = ta3.reshape(_V, 2 * _A)
    td = td3.reshape(_V, 8)
    fi, dv, part = _k1()(index_h, index_t.T, td)
    out_t = _k2()(fi, dv, idx_t.T, ta, part)
    return out_t.T


# speedup vs baseline: 30.1723x; 1.2352x over previous
import functools

import jax
import jax.numpy as jnp
from jax import lax
from jax.experimental import pallas as pl
from jax.experimental.pallas import tpu as pltpu
from jax.experimental.pallas import tpu_sc as plsc

_B, _L, _A = 4096, 64, 4
_H, _W = 50000, 32
_LP = _L + 1
_V = _B * _LP
_NC, _NS = 2, 16
_NW = _NC * _NS
_CHUNK = 1568
_RB = 16
_D1 = 704
_D2 = 736
_DOUT = 992


_TBLK = 64


def _tables_body(a_ref, d_ref, ta_ref, td_ref):
    a = a_ref[...]
    ta_ref[:, 0:_L, 0:_A] = jnp.sin(a)
    ta_ref[:, 0:_L, _A:2 * _A] = jnp.cos(a)
    ta_ref[:, _L:_LP, :] = jnp.zeros((_TBLK, 1, 2 * _A), jnp.float32)
    td_ref[...] = jnp.zeros((_TBLK, _LP, 8), jnp.float32)
    td_ref[:, 0:_L, 0:1] = d_ref[...][:, :, None]


def _build_tables(dist, angle):
    grid = _B // _TBLK
    return pl.pallas_call(
        _tables_body,
        grid=(grid,),
        in_specs=[
            pl.BlockSpec((_TBLK, _L, _A), lambda i: (i, 0, 0)),
            pl.BlockSpec((_TBLK, _L), lambda i: (i, 0)),
        ],
        out_specs=[
            pl.BlockSpec((_TBLK, _LP, 2 * _A), lambda i: (i, 0, 0)),
            pl.BlockSpec((_TBLK, _LP, 8), lambda i: (i, 0, 0)),
        ],
        out_shape=[
            jax.ShapeDtypeStruct((_B, _LP, 2 * _A), jnp.float32),
            jax.ShapeDtypeStruct((_B, _LP, 8), jnp.float32),
        ],
    )(angle, dist)


@functools.lru_cache(maxsize=None)
def _mesh():
    return plsc.VectorSubcoreMesh(
        core_axis_name="c", subcore_axis_name="s",
        num_cores=_NC, num_subcores=_NS,
    )


def _worker_id():
    return lax.axis_index("s") * _NC + lax.axis_index("c")


def _k1_body(ih_hbm, itT_hbm, td_hbm, fi_out, dv_out, part_out,
             ihB, itB, fiB, dbB, dvB, stage, sem, semo):
    wid = _worker_id()
    r_start = wid * _CHUNK
    nrows = jnp.minimum(_CHUNK, _H - r_start)
    nblk = nrows // _RB
    iota16 = lax.iota(jnp.int32, 16)
    zcol = jnp.zeros((16,), jnp.int32)

    def blk(b, carry):
        mn, mx = carry
        r0 = r_start + b * _RB
        c1 = pltpu.async_copy(ih_hbm.at[pl.ds(r0, _RB)], ihB, sem)
        c2 = pltpu.async_copy(itT_hbm.at[:, pl.ds(r0, _RB)], itB, sem)
        @pl.when(b > 0)
        def _():
            pltpu.make_async_copy(fiB, fi_out.at[pl.ds(0, 4)], semo).wait()
            pltpu.make_async_copy(dvB, dv_out.at[pl.ds(0, _RB * _W)],
                                  semo).wait()
        c1.wait()
        c2.wait()
        ihv = ihB[...] * _LP
        for w in range(_W):
            v = itB[w, :] + ihv
            fiB[w // 8, pl.ds(16 * (w % 8), 16)] = v
        descs = [
            pltpu.async_copy(td_hbm.at[fiB.at[g]],
                             dbB.at[pl.ds(128 * g, 128)], sem)
            for g in range(4)
        ]
        for dsc in descs:
            dsc.wait()
        for w in range(_W):
            vals = plsc.load_gather(dbB, [16 * w + iota16, zcol])
            dvB[pl.ds(16 * w, 16)] = vals
            mn = jnp.minimum(mn, vals)
            mx = jnp.maximum(mx, vals)
        pltpu.async_copy(fiB, fi_out.at[pl.ds(r0 // 4, 4)], semo)
        pltpu.async_copy(dvB, dv_out.at[pl.ds(r0 * _W, _RB * _W)], semo)
        return mn, mx

    big = jnp.full((16,), 3.0e38, jnp.float32)
    mn, mx = lax.fori_loop(0, nblk, blk, (big, -big))
    pltpu.make_async_copy(fiB, fi_out.at[pl.ds(0, 4)], semo).wait()
    pltpu.make_async_copy(dvB, dv_out.at[pl.ds(0, _RB * _W)], semo).wait()
    stage[pl.ds(0, 16)] = mn
    stage[pl.ds(16, 16)] = mx
    pltpu.sync_copy(stage, part_out.at[pl.ds(32 * wid, 32)])


@functools.lru_cache(maxsize=None)
def _k1():
    return pl.kernel(
        _k1_body,
        out_type=(
            jax.ShapeDtypeStruct((_H * _W // 128, 128), jnp.int32),
            jax.ShapeDtypeStruct((_H * _W,), jnp.float32),
            jax.ShapeDtypeStruct((_NW * 32,), jnp.float32),
        ),
        mesh=_mesh(),
        compiler_params=pltpu.CompilerParams(needs_layout_passes=False, use_tc_tiling_on_sc=False),
        scratch_types=[
            pltpu.VMEM((_RB,), jnp.int32),
            pltpu.VMEM((_W, _RB), jnp.int32),
            pltpu.VMEM((4, 128), jnp.int32),
            pltpu.VMEM((_RB * _W, 8), jnp.float32),
            pltpu.VMEM((_RB * _W,), jnp.float32),
            pltpu.VMEM((32,), jnp.float32),
            pltpu.SemaphoreType.DMA,
            pltpu.SemaphoreType.DMA,
        ],
    )


def _k2_body(fi_hbm, dv_hbm, idtT_hbm, ta_hbm, part_hbm, outT_hbm,
             fiB0, fiB1, dvB0, dvB1, idB0, idB1, abB0, abB1, xd0, xd1,
             pbuf, semi0, semi1, semg0, semg1, semo0, semo1):
    wid = _worker_id()
    r_start = wid * _CHUNK
    nrows = jnp.minimum(_CHUNK, _H - r_start)
    nblk = nrows // _RB
    iota16 = lax.iota(jnp.int32, 16)
    ones16 = jnp.ones((16,), jnp.float32)
    zero16 = jnp.zeros((16,), jnp.float32)

    pltpu.sync_copy(part_hbm, pbuf)
    mn = pbuf[pl.ds(0, 16)]
    mx = pbuf[pl.ds(16, 16)]
    for i in range(1, _NW):
        mn = jnp.minimum(mn, pbuf[pl.ds(32 * i, 16)])
        mx = jnp.maximum(mx, pbuf[pl.ds(32 * i + 16, 16)])
    gmin = jnp.min(mn)
    inv_v = ones16 / (jnp.full((16,), 1.0, jnp.float32) * (jnp.max(mx) - gmin))

    def process(b, fiB, dvB, idB, abB, xd, semi, semg, semo):
        r0 = r_start + b * _RB
        c1 = pltpu.async_copy(fi_hbm.at[pl.ds(r0 // 4, 4)], fiB, semi)
        c2 = pltpu.async_copy(dv_hbm.at[pl.ds(r0 * _W, _RB * _W)], dvB, semi)

        @pl.when(b >= 2)
        def _():
            pltpu.make_async_copy(xd, outT_hbm.at[:, pl.ds(0, _RB)],
                                  semo).wait()
            for w in range(_W):
                plsc.store_scatter(xd, [22 * w + idB[w, :], iota16], zero16)

        @pl.when(b < 2)
        def _():
            for c in range(_D1):
                xd[c, :] = zero16

        c3 = pltpu.async_copy(idtT_hbm.at[:, pl.ds(r0, _RB)], idB, semi)
        c1.wait()
        descs = [
            pltpu.async_copy(ta_hbm.at[fiB.at[g]],
                             abB.at[pl.ds(128 * g, 128)], semg)
            for g in range(4)
        ]
        c2.wait()
        for w in range(_W):
            d = dvB[pl.ds(16 * w, 16)]
            xd[_D1 + w, :] = (d - gmin) * inv_v
        c3.wait()
        for w in range(_W):
            plsc.store_scatter(xd, [22 * w + idB[w, :], iota16], ones16)
        for dsc in descs:
            dsc.wait()
        for w in range(_W):
            base = 16 * w + iota16
            for j in range(8):
                v = plsc.load_gather(abB, [base, jnp.full((16,), j, jnp.int32)])
                xd[_D2 + 8 * w + j, :] = v
        pltpu.async_copy(xd, outT_hbm.at[:, pl.ds(r0, _RB)], semo)

    def blk(b, _):
        @pl.when(b % 2 == 0)
        def _():
            process(b, fiB0, dvB0, idB0, abB0, xd0, semi0, semg0, semo0)

        @pl.when(b % 2 == 1)
        def _():
            process(b, fiB1, dvB1, idB1, abB1, xd1, semi1, semg1, semo1)

        return 0

    lax.fori_loop(0, nblk, blk, 0)
    pltpu.make_async_copy(xd0, outT_hbm.at[:, pl.ds(0, _RB)], semo0).wait()
    pltpu.make_async_copy(xd1, outT_hbm.at[:, pl.ds(0, _RB)], semo1).wait()


@functools.lru_cache(maxsize=None)
def _k2():
    return pl.kernel(
        _k2_body,
        out_type=jax.ShapeDtypeStruct((_DOUT, _H), jnp.float32),
        mesh=_mesh(),
        compiler_params=pltpu.CompilerParams(needs_layout_passes=False, use_tc_tiling_on_sc=False),
        scratch_types=[
            pltpu.VMEM((4, 128), jnp.int32),
            pltpu.VMEM((4, 128), jnp.int32),
            pltpu.VMEM((_RB * _W,), jnp.float32),
            pltpu.VMEM((_RB * _W,), jnp.float32),
            pltpu.VMEM((_W, _RB), jnp.int32),
            pltpu.VMEM((_W, _RB), jnp.int32),
            pltpu.VMEM((_RB * _W, 8), jnp.float32),
            pltpu.VMEM((_RB * _W, 8), jnp.float32),
            pltpu.VMEM((_DOUT, _RB), jnp.float32),
            pltpu.VMEM((_DOUT, _RB), jnp.float32),
            pltpu.VMEM((_NW * 32,), jnp.float32),
            pltpu.SemaphoreType.DMA,
            pltpu.SemaphoreType.DMA,
            pltpu.SemaphoreType.DMA,
            pltpu.SemaphoreType.DMA,
            pltpu.SemaphoreType.DMA,
            pltpu.SemaphoreType.DMA,
        ],
    )


def kernel(dist, angle, idx_t, index_t, index_h):
    ta3, td3 = _build_tables(dist, angle)
    ta = ta3.reshape(_V, 2 * _A)
    td = td3.reshape(_V, 8)
    fi, dv, part = _k1()(index_h, index_t.T, td)
    out_t = _k2()(fi, dv, idx_t.T, ta, part)
    return out_t.T


# speedup vs baseline: 50.6371x; 1.6783x over previous
import functools

import jax
import jax.numpy as jnp
from jax import lax
from jax.experimental import pallas as pl
from jax.experimental.pallas import tpu as pltpu
from jax.experimental.pallas import tpu_sc as plsc

_B, _L, _A = 4096, 64, 4
_H, _W = 50000, 32
_LP = _L + 1
_V = _B * _LP
_NC, _NS = 2, 16
_NW = _NC * _NS
_CHUNK = 1568
_RB = 16
_D1 = 704
_D2 = 736
_DOUT = 992


def _tables_body(a_ref, d_ref, ta_ref, td_ref):
    i = pl.program_id(0)

    @pl.when(i < _L)
    def _():
        a = a_ref[0]
        sc = jnp.concatenate([jnp.sin(a), jnp.cos(a)], axis=0)
        ta_ref[0] = jnp.transpose(sc, (1, 0))
        td_ref[...] = d_ref[...]

    @pl.when(i == _L)
    def _():
        ta_ref[...] = jnp.zeros((1, _B, 2 * _A), jnp.float32)
        td_ref[...] = jnp.zeros((1, 1, _B), jnp.float32)


def _build_tables(dist, angle):
    at = jnp.transpose(angle, (1, 2, 0))
    dt = dist.T[:, None, :]
    return pl.pallas_call(
        _tables_body,
        grid=(_LP,),
        in_specs=[
            pl.BlockSpec((1, _A, _B), lambda i: (jnp.minimum(i, _L - 1), 0, 0)),
            pl.BlockSpec((1, 1, _B), lambda i: (jnp.minimum(i, _L - 1), 0, 0)),
        ],
        out_specs=[
            pl.BlockSpec((1, _B, 2 * _A), lambda i: (i, 0, 0)),
            pl.BlockSpec((1, 1, _B), lambda i: (i, 0, 0)),
        ],
        out_shape=[
            jax.ShapeDtypeStruct((_LP, _B, 2 * _A), jnp.float32),
            jax.ShapeDtypeStruct((_LP, 1, _B), jnp.float32),
        ],
    )(at, dt)


@functools.lru_cache(maxsize=None)
def _mesh():
    return plsc.VectorSubcoreMesh(
        core_axis_name="c", subcore_axis_name="s",
        num_cores=_NC, num_subcores=_NS,
    )


def _worker_id():
    return lax.axis_index("s") * _NC + lax.axis_index("c")


def _k1_body(ih_hbm, itT_hbm, td_hbm, fi_out, dv_out, part_out,
             ihB, itB, fiB, dvB, stage, sem, semo):
    wid = _worker_id()
    r_start = wid * _CHUNK
    nrows = jnp.minimum(_CHUNK, _H - r_start)
    nblk = nrows // _RB
    def blk(b, carry):
        mn, mx = carry
        r0 = r_start + b * _RB
        c1 = pltpu.async_copy(ih_hbm.at[pl.ds(r0, _RB)], ihB, sem)
        c2 = pltpu.async_copy(itT_hbm.at[:, pl.ds(r0, _RB)], itB, sem)
        @pl.when(b > 0)
        def _():
            pltpu.make_async_copy(fiB, fi_out.at[pl.ds(0, 4)], semo).wait()
            pltpu.make_async_copy(dvB, dv_out.at[pl.ds(0, _RB * _W)],
                                  semo).wait()
        c1.wait()
        c2.wait()
        ihv = ihB[...]
        for w in range(_W):
            v = itB[w, :] * _B + ihv
            fiB[w // 8, pl.ds(16 * (w % 8), 16)] = v
        descs = [
            pltpu.async_copy(td_hbm.at[fiB.at[g]],
                             dvB.at[pl.ds(128 * g, 128)], sem)
            for g in range(4)
        ]
        for dsc in descs:
            dsc.wait()
        for w in range(_W):
            vals = dvB[pl.ds(16 * w, 16)]
            mn = jnp.minimum(mn, vals)
            mx = jnp.maximum(mx, vals)
        pltpu.async_copy(fiB, fi_out.at[pl.ds(r0 // 4, 4)], semo)
        pltpu.async_copy(dvB, dv_out.at[pl.ds(r0 * _W, _RB * _W)], semo)
        return mn, mx

    big = jnp.full((16,), 3.0e38, jnp.float32)
    mn, mx = lax.fori_loop(0, nblk, blk, (big, -big))
    pltpu.make_async_copy(fiB, fi_out.at[pl.ds(0, 4)], semo).wait()
    pltpu.make_async_copy(dvB, dv_out.at[pl.ds(0, _RB * _W)], semo).wait()
    stage[pl.ds(0, 16)] = mn
    stage[pl.ds(16, 16)] = mx
    pltpu.sync_copy(stage, part_out.at[pl.ds(32 * wid, 32)])


@functools.lru_cache(maxsize=None)
def _k1():
    return pl.kernel(
        _k1_body,
        out_type=(
            jax.ShapeDtypeStruct((_H * _W // 128, 128), jnp.int32),
            jax.ShapeDtypeStruct((_H * _W,), jnp.float32),
            jax.ShapeDtypeStruct((_NW * 32,), jnp.float32),
        ),
        mesh=_mesh(),
        compiler_params=pltpu.CompilerParams(needs_layout_passes=False, use_tc_tiling_on_sc=False),
        scratch_types=[
            pltpu.VMEM((_RB,), jnp.int32),
            pltpu.VMEM((_W, _RB), jnp.int32),
            pltpu.VMEM((4, 128), jnp.int32),
            pltpu.VMEM((_RB * _W,), jnp.float32),
            pltpu.VMEM((32,), jnp.float32),
            pltpu.SemaphoreType.DMA,
            pltpu.SemaphoreType.DMA,
        ],
    )


def _k2_body(fi_hbm, dv_hbm, idtT_hbm, ta_hbm, part_hbm, outT_hbm,
             fiB0, fiB1, dvB0, dvB1, idB0, idB1, abB0, abB1, xd0, xd1,
             pbuf, semi0, semi1, semg0, semg1, semo0, semo1):
    wid = _worker_id()
    r_start = wid * _CHUNK
    nrows = jnp.minimum(_CHUNK, _H - r_start)
    nblk = nrows // _RB
    iota16 = lax.iota(jnp.int32, 16)
    ones16 = jnp.ones((16,), jnp.float32)
    zero16 = jnp.zeros((16,), jnp.float32)

    pltpu.sync_copy(part_hbm, pbuf)
    mn = pbuf[pl.ds(0, 16)]
    mx = pbuf[pl.ds(16, 16)]
    for i in range(1, _NW):
        mn = jnp.minimum(mn, pbuf[pl.ds(32 * i, 16)])
        mx = jnp.maximum(mx, pbuf[pl.ds(32 * i + 16, 16)])
    gmin = jnp.min(mn)
    inv_v = ones16 / (jnp.full((16,), 1.0, jnp.float32) * (jnp.max(mx) - gmin))

    def process(b, fiB, dvB, idB, abB, xd, semi, semg, semo):
        r0 = r_start + b * _RB
        c1 = pltpu.async_copy(fi_hbm.at[pl.ds(r0 // 4, 4)], fiB, semi)
        c2 = pltpu.async_copy(dv_hbm.at[pl.ds(r0 * _W, _RB * _W)], dvB, semi)

        @pl.when(b >= 2)
        def _():
            pltpu.make_async_copy(xd, outT_hbm.at[:, pl.ds(0, _RB)],
                                  semo).wait()
            for w in range(_W):
                plsc.store_scatter(xd, [22 * w + idB[w, :], iota16], zero16)

        @pl.when(b < 2)
        def _():
            for c in range(_D1):
                xd[c, :] = zero16

        c3 = pltpu.async_copy(idtT_hbm.at[:, pl.ds(r0, _RB)], idB, semi)
        c1.wait()
        descs = [
            pltpu.async_copy(ta_hbm.at[fiB.at[g]],
                             abB.at[pl.ds(128 * g, 128)], semg)
            for g in range(4)
        ]
        c2.wait()
        for w in range(_W):
            d = dvB[pl.ds(16 * w, 16)]
            xd[_D1 + w, :] = (d - gmin) * inv_v
        c3.wait()
        for w in range(_W):
            plsc.store_scatter(xd, [22 * w + idB[w, :], iota16], ones16)
        for dsc in descs:
            dsc.wait()
        for w in range(_W):
            base = 16 * w + iota16
            for j in range(8):
                v = plsc.load_gather(abB, [base, jnp.full((16,), j, jnp.int32)])
                xd[_D2 + 8 * w + j, :] = v
        pltpu.async_copy(xd, outT_hbm.at[:, pl.ds(r0, _RB)], semo)

    def blk(b, _):
        @pl.when(b % 2 == 0)
        def _():
            process(b, fiB0, dvB0, idB0, abB0, xd0, semi0, semg0, semo0)

        @pl.when(b % 2 == 1)
        def _():
            process(b, fiB1, dvB1, idB1, abB1, xd1, semi1, semg1, semo1)

        return 0

    lax.fori_loop(0, nblk, blk, 0)
    pltpu.make_async_copy(xd0, outT_hbm.at[:, pl.ds(0, _RB)], semo0).wait()
    pltpu.make_async_copy(xd1, outT_hbm.at[:, pl.ds(0, _RB)], semo1).wait()


@functools.lru_cache(maxsize=None)
def _k2():
    return pl.kernel(
        _k2_body,
        out_type=jax.ShapeDtypeStruct((_DOUT, _H), jnp.float32),
        mesh=_mesh(),
        compiler_params=pltpu.CompilerParams(needs_layout_passes=False, use_tc_tiling_on_sc=False),
        scratch_types=[
            pltpu.VMEM((4, 128), jnp.int32),
            pltpu.VMEM((4, 128), jnp.int32),
            pltpu.VMEM((_RB * _W,), jnp.float32),
            pltpu.VMEM((_RB * _W,), jnp.float32),
            pltpu.VMEM((_W, _RB), jnp.int32),
            pltpu.VMEM((_W, _RB), jnp.int32),
            pltpu.VMEM((_RB * _W, 8), jnp.float32),
            pltpu.VMEM((_RB * _W, 8), jnp.float32),
            pltpu.VMEM((_DOUT, _RB), jnp.float32),
            pltpu.VMEM((_DOUT, _RB), jnp.float32),
            pltpu.VMEM((_NW * 32,), jnp.float32),
            pltpu.SemaphoreType.DMA,
            pltpu.SemaphoreType.DMA,
            pltpu.SemaphoreType.DMA,
            pltpu.SemaphoreType.DMA,
            pltpu.SemaphoreType.DMA,
            pltpu.SemaphoreType.DMA,
        ],
    )


def kernel(dist, angle, idx_t, index_t, index_h):
    ta3, td2 = _build_tables(dist, angle)
    ta = ta3.reshape(_V, 2 * _A)
    td = td2.reshape(_V)
    fi, dv, part = _k1()(index_h, index_t.T, td)
    out_t = _k2()(fi, dv, idx_t.T, ta, part)
    return out_t.T


# speedup vs baseline: 58.6998x; 1.1592x over previous
import functools

import jax
import jax.numpy as jnp
from jax import lax
from jax.experimental import pallas as pl
from jax.experimental.pallas import tpu as pltpu
from jax.experimental.pallas import tpu_sc as plsc

_B, _L, _A = 4096, 64, 4
_H, _W = 50000, 32
_LP = _L + 1
_V = _B * _LP
_NC, _NS = 2, 16
_NW = _NC * _NS
_CHUNK = 1568
_RB = 16
_D1 = 704
_D2 = 736
_DOUT = 992


def _tables_body(a_ref, d_ref, ta_ref, td_ref):
    i = pl.program_id(0)

    @pl.when(i < _L // 8)
    def _():
        a = a_ref[...]
        sc = jnp.concatenate([jnp.sin(a), jnp.cos(a)], axis=1)
        ta_ref[...] = jnp.transpose(sc, (0, 2, 1))
        td_ref[...] = d_ref[...]

    @pl.when(i == _L // 8)
    def _():
        ta_ref[...] = jnp.zeros((8, _B, 2 * _A), jnp.float32)
        td_ref[...] = jnp.zeros((8, 1, _B), jnp.float32)


def _build_tables(dist, angle):
    at = jnp.transpose(angle, (1, 2, 0))
    dt = dist.T[:, None, :]
    nb = _L // 8
    return pl.pallas_call(
        _tables_body,
        grid=(nb + 1,),
        in_specs=[
            pl.BlockSpec((8, _A, _B), lambda i: (jnp.minimum(i, nb - 1), 0, 0)),
            pl.BlockSpec((8, 1, _B), lambda i: (jnp.minimum(i, nb - 1), 0, 0)),
        ],
        out_specs=[
            pl.BlockSpec((8, _B, 2 * _A), lambda i: (i, 0, 0)),
            pl.BlockSpec((8, 1, _B), lambda i: (i, 0, 0)),
        ],
        out_shape=[
            jax.ShapeDtypeStruct((_LP, _B, 2 * _A), jnp.float32),
            jax.ShapeDtypeStruct((_LP, 1, _B), jnp.float32),
        ],
    )(at, dt)


@functools.lru_cache(maxsize=None)
def _mesh():
    return plsc.VectorSubcoreMesh(
        core_axis_name="c", subcore_axis_name="s",
        num_cores=_NC, num_subcores=_NS,
    )


def _worker_id():
    return lax.axis_index("s") * _NC + lax.axis_index("c")


def _k1_body(ih_hbm, itT_hbm, td_hbm, fi_out, dv_out, part_out,
             ihB, itB, fiB, dvB, stage, sem, semo):
    wid = _worker_id()
    r_start = wid * _CHUNK
    nrows = jnp.minimum(_CHUNK, _H - r_start)
    nblk = nrows // _RB
    pltpu.async_copy(ih_hbm.at[pl.ds(r_start, _RB)], ihB, sem)
    pltpu.async_copy(itT_hbm.at[:, pl.ds(r_start, _RB)], itB, sem)

    def blk(b, carry):
        mn, mx = carry
        r0 = r_start + b * _RB
        pltpu.make_async_copy(ih_hbm.at[pl.ds(0, _RB)], ihB, sem).wait()
        pltpu.make_async_copy(itT_hbm.at[:, pl.ds(0, _RB)], itB, sem).wait()
        @pl.when(b > 0)
        def _():
            pltpu.make_async_copy(fiB, fi_out.at[pl.ds(0, 4)], semo).wait()
            pltpu.make_async_copy(dvB, dv_out.at[pl.ds(0, _RB * _W)],
                                  semo).wait()
        ihv = ihB[...]
        for w in range(_W):
            v = itB[w, :] * _B + ihv
            fiB[w // 8, pl.ds(16 * (w % 8), 16)] = v

        @pl.when(b + 1 < nblk)
        def _():
            r1 = r0 + _RB
            pltpu.async_copy(ih_hbm.at[pl.ds(r1, _RB)], ihB, sem)
            pltpu.async_copy(itT_hbm.at[:, pl.ds(r1, _RB)], itB, sem)

        descs = [
            pltpu.async_copy(td_hbm.at[fiB.at[g]],
                             dvB.at[pl.ds(128 * g, 128)], sem)
            for g in range(4)
        ]
        for dsc in descs:
            dsc.wait()
        for w in range(_W):
            vals = dvB[pl.ds(16 * w, 16)]
            mn = jnp.minimum(mn, vals)
            mx = jnp.maximum(mx, vals)
        pltpu.async_copy(fiB, fi_out.at[pl.ds(r0 // 4, 4)], semo)
        pltpu.async_copy(dvB, dv_out.at[pl.ds(r0 * _W, _RB * _W)], semo)
        return mn, mx

    big = jnp.full((16,), 3.0e38, jnp.float32)
    mn, mx = lax.fori_loop(0, nblk, blk, (big, -big))
    pltpu.make_async_copy(fiB, fi_out.at[pl.ds(0, 4)], semo).wait()
    pltpu.make_async_copy(dvB, dv_out.at[pl.ds(0, _RB * _W)], semo).wait()
    stage[pl.ds(0, 16)] = mn
    stage[pl.ds(16, 16)] = mx
    pltpu.sync_copy(stage, part_out.at[pl.ds(32 * wid, 32)])


@functools.lru_cache(maxsize=None)
def _k1():
    return pl.kernel(
        _k1_body,
        out_type=(
            jax.ShapeDtypeStruct((_H * _W // 128, 128), jnp.int32),
            jax.ShapeDtypeStruct((_H * _W,), jnp.float32),
            jax.ShapeDtypeStruct((_NW * 32,), jnp.float32),
        ),
        mesh=_mesh(),
        compiler_params=pltpu.CompilerParams(needs_layout_passes=False, use_tc_tiling_on_sc=False),
        scratch_types=[
            pltpu.VMEM((_RB,), jnp.int32),
            pltpu.VMEM((_W, _RB), jnp.int32),
            pltpu.VMEM((4, 128), jnp.int32),
            pltpu.VMEM((_RB * _W,), jnp.float32),
            pltpu.VMEM((32,), jnp.float32),
            pltpu.SemaphoreType.DMA,
            pltpu.SemaphoreType.DMA,
        ],
    )


def _k2_body(fi_hbm, dv_hbm, idtT_hbm, ta_hbm, part_hbm, outT_hbm,
             fiB0, fiB1, dvB0, dvB1, idB0, idB1, abB0, abB1, xd0, xd1,
             posB0, posB1, pbuf, semi0, semi1, semg0, semg1, semo0, semo1):
    wid = _worker_id()
    r_start = wid * _CHUNK
    nrows = jnp.minimum(_CHUNK, _H - r_start)
    nblk = nrows // _RB
    iota16 = lax.iota(jnp.int32, 16)
    ones16 = jnp.ones((16,), jnp.float32)
    zero16 = jnp.zeros((16,), jnp.float32)

    pltpu.sync_copy(part_hbm, pbuf)
    mn = pbuf[pl.ds(0, 16)]
    mx = pbuf[pl.ds(16, 16)]
    for i in range(1, _NW):
        mn = jnp.minimum(mn, pbuf[pl.ds(32 * i, 16)])
        mx = jnp.maximum(mx, pbuf[pl.ds(32 * i + 16, 16)])
    gmin = jnp.min(mn)
    inv_v = ones16 / (jnp.full((16,), 1.0, jnp.float32) * (jnp.max(mx) - gmin))

    def inload(b, fiB, dvB, idB, semi):
        r0 = r_start + b * _RB
        pltpu.async_copy(fi_hbm.at[pl.ds(r0 // 4, 4)], fiB, semi)
        pltpu.async_copy(dv_hbm.at[pl.ds(r0 * _W, _RB * _W)], dvB, semi)
        pltpu.async_copy(idtT_hbm.at[:, pl.ds(r0, _RB)], idB, semi)

    def process(b, fiB, dvB, idB, abB, xd, posB, semi, semg, semo,
                nfiB, ndvB, nidB, nsemi):
        r0 = r_start + b * _RB
        pltpu.make_async_copy(fi_hbm.at[pl.ds(0, 4)], fiB, semi).wait()
        pltpu.make_async_copy(dv_hbm.at[pl.ds(0, _RB * _W)], dvB, semi).wait()
        pltpu.make_async_copy(idtT_hbm.at[:, pl.ds(0, _RB)], idB, semi).wait()
        descs = [
            pltpu.async_copy(ta_hbm.at[fiB.at[g]],
                             abB.at[pl.ds(128 * g, 128)], semg)
            for g in range(4)
        ]

        @pl.when(b + 1 < nblk)
        def _():
            inload(b + 1, nfiB, ndvB, nidB, nsemi)

        @pl.when(b >= 2)
        def _():
            pltpu.make_async_copy(xd, outT_hbm.at[:, pl.ds(0, _RB)],
                                  semo).wait()
            for w in range(_W):
                plsc.store_scatter(xd, [posB[w, :], iota16], zero16)

        @pl.when(b < 2)
        def _():
            for c in range(_D1):
                xd[c, :] = zero16

        for w in range(_W):
            d = dvB[pl.ds(16 * w, 16)]
            xd[_D1 + w, :] = (d - gmin) * inv_v
        for w in range(_W):
            pos = 22 * w + idB[w, :]
            plsc.store_scatter(xd, [pos, iota16], ones16)
            posB[w, :] = pos
        for dsc in descs:
            dsc.wait()
        for w in range(_W):
            base = 16 * w + iota16
            for j in range(8):
                v = plsc.load_gather(abB, [base, jnp.full((16,), j, jnp.int32)])
                xd[_D2 + 8 * w + j, :] = v
        pltpu.async_copy(xd, outT_hbm.at[:, pl.ds(r0, _RB)], semo)

    inload(0, fiB0, dvB0, idB0, semi0)

    def blk(b, _):
        @pl.when(b % 2 == 0)
        def _():
            process(b, fiB0, dvB0, idB0, abB0, xd0, posB0, semi0, semg0,
                    semo0, fiB1, dvB1, idB1, semi1)

        @pl.when(b % 2 == 1)
        def _():
            process(b, fiB1, dvB1, idB1, abB1, xd1, posB1, semi1, semg1,
                    semo1, fiB0, dvB0, idB0, semi0)

        return 0

    lax.fori_loop(0, nblk, blk, 0)
    pltpu.make_async_copy(xd0, outT_hbm.at[:, pl.ds(0, _RB)], semo0).wait()
    pltpu.make_async_copy(xd1, outT_hbm.at[:, pl.ds(0, _RB)], semo1).wait()


@functools.lru_cache(maxsize=None)
def _k2():
    return pl.kernel(
        _k2_body,
        out_type=jax.ShapeDtypeStruct((_DOUT, _H), jnp.float32),
        mesh=_mesh(),
        compiler_params=pltpu.CompilerParams(needs_layout_passes=False, use_tc_tiling_on_sc=False),
        scratch_types=[
            pltpu.VMEM((4, 128), jnp.int32),
            pltpu.VMEM((4, 128), jnp.int32),
            pltpu.VMEM((_RB * _W,), jnp.float32),
            pltpu.VMEM((_RB * _W,), jnp.float32),
            pltpu.VMEM((_W, _RB), jnp.int32),
            pltpu.VMEM((_W, _RB), jnp.int32),
            pltpu.VMEM((_RB * _W, 8), jnp.float32),
            pltpu.VMEM((_RB * _W, 8), jnp.float32),
            pltpu.VMEM((_DOUT, _RB), jnp.float32),
            pltpu.VMEM((_DOUT, _RB), jnp.float32),
            pltpu.VMEM((_W, 16), jnp.int32),
            pltpu.VMEM((_W, 16), jnp.int32),
            pltpu.VMEM((_NW * 32,), jnp.float32),
            pltpu.SemaphoreType.DMA,
            pltpu.SemaphoreType.DMA,
            pltpu.SemaphoreType.DMA,
            pltpu.SemaphoreType.DMA,
            pltpu.SemaphoreType.DMA,
            pltpu.SemaphoreType.DMA,
        ],
    )


def kernel(dist, angle, idx_t, index_t, index_h):
    ta3, td2 = _build_tables(dist, angle)
    ta = ta3.reshape(_V, 2 * _A)
    td = td2.reshape(_V)
    fi, dv, part = _k1()(index_h, index_t.T, td)
    out_t = _k2()(fi, dv, idx_t.T, ta, part)
    return out_t.T


# speedup vs baseline: 58.9075x; 1.0035x over previous
import functools

import jax
import jax.numpy as jnp
from jax import lax
from jax.experimental import pallas as pl
from jax.experimental.pallas import tpu as pltpu
from jax.experimental.pallas import tpu_sc as plsc

_B, _L, _A = 4096, 64, 4
_H, _W = 50000, 32
_LP = _L + 1
_V = _B * _LP
_NC, _NS = 2, 16
_NW = _NC * _NS
_CHUNK = 1568
_RB = 16
_D1 = 704
_D2 = 736
_DOUT = 992


def _tables_body(a_ref, d_ref, ta_ref, td_ref):
    i = pl.program_id(0)

    @pl.when(i < _L // 8)
    def _():
        a = a_ref[...]
        sc = jnp.concatenate([jnp.sin(a), jnp.cos(a)], axis=1)
        ta_ref[...] = jnp.transpose(sc, (0, 2, 1))
        td_ref[...] = d_ref[...]

    @pl.when(i == _L // 8)
    def _():
        ta_ref[...] = jnp.zeros((8, _B, 2 * _A), jnp.float32)
        td_ref[...] = jnp.zeros((8, 1, _B), jnp.float32)


def _build_tables(dist, angle):
    at = jnp.transpose(angle, (1, 2, 0))
    dt = dist.T[:, None, :]
    nb = _L // 8
    return pl.pallas_call(
        _tables_body,
        grid=(nb + 1,),
        in_specs=[
            pl.BlockSpec((8, _A, _B), lambda i: (jnp.minimum(i, nb - 1), 0, 0)),
            pl.BlockSpec((8, 1, _B), lambda i: (jnp.minimum(i, nb - 1), 0, 0)),
        ],
        out_specs=[
            pl.BlockSpec((8, _B, 2 * _A), lambda i: (i, 0, 0)),
            pl.BlockSpec((8, 1, _B), lambda i: (i, 0, 0)),
        ],
        out_shape=[
            jax.ShapeDtypeStruct((_LP, _B, 2 * _A), jnp.float32),
            jax.ShapeDtypeStruct((_LP, 1, _B), jnp.float32),
        ],
    )(at, dt)


@functools.lru_cache(maxsize=None)
def _mesh():
    return plsc.VectorSubcoreMesh(
        core_axis_name="c", subcore_axis_name="s",
        num_cores=_NC, num_subcores=_NS,
    )


def _worker_id():
    return lax.axis_index("s") * _NC + lax.axis_index("c")


def _k1_body(ih_hbm, itT_hbm, td_hbm, fi_out, dv_out, part_out,
             ihB, itB, fiB, dvB, stage, seml, sem, semo):
    wid = _worker_id()
    r_start = wid * _CHUNK
    nrows = jnp.minimum(_CHUNK, _H - r_start)
    nblk = nrows // _RB
    pltpu.async_copy(ih_hbm.at[pl.ds(r_start, _RB)], ihB, seml)
    pltpu.async_copy(itT_hbm.at[:, pl.ds(r_start, _RB)], itB, seml)

    def blk(b, carry):
        mn, mx = carry
        r0 = r_start + b * _RB
        pltpu.make_async_copy(ih_hbm.at[pl.ds(0, _RB)], ihB, seml).wait()
        pltpu.make_async_copy(itT_hbm.at[:, pl.ds(0, _RB)], itB, seml).wait()
        @pl.when(b > 0)
        def _():
            pltpu.make_async_copy(fiB, fi_out.at[pl.ds(0, 4)], semo).wait()
            pltpu.make_async_copy(dvB, dv_out.at[pl.ds(0, _RB * _W)],
                                  semo).wait()
        ihv = ihB[...]
        for w in range(_W):
            v = itB[w, :] * _B + ihv
            fiB[w // 8, pl.ds(16 * (w % 8), 16)] = v

        @pl.when(b + 1 < nblk)
        def _():
            r1 = r0 + _RB
            pltpu.async_copy(ih_hbm.at[pl.ds(r1, _RB)], ihB, seml)
            pltpu.async_copy(itT_hbm.at[:, pl.ds(r1, _RB)], itB, seml)

        descs = [
            pltpu.async_copy(td_hbm.at[fiB.at[g]],
                             dvB.at[pl.ds(128 * g, 128)], sem)
            for g in range(4)
        ]
        for dsc in descs:
            dsc.wait()
        for w in range(_W):
            vals = dvB[pl.ds(16 * w, 16)]
            mn = jnp.minimum(mn, vals)
            mx = jnp.maximum(mx, vals)
        pltpu.async_copy(fiB, fi_out.at[pl.ds(r0 // 4, 4)], semo)
        pltpu.async_copy(dvB, dv_out.at[pl.ds(r0 * _W, _RB * _W)], semo)
        return mn, mx

    big = jnp.full((16,), 3.0e38, jnp.float32)
    mn, mx = lax.fori_loop(0, nblk, blk, (big, -big))
    pltpu.make_async_copy(fiB, fi_out.at[pl.ds(0, 4)], semo).wait()
    pltpu.make_async_copy(dvB, dv_out.at[pl.ds(0, _RB * _W)], semo).wait()
    stage[pl.ds(0, 16)] = mn
    stage[pl.ds(16, 16)] = mx
    pltpu.sync_copy(stage, part_out.at[pl.ds(32 * wid, 32)])


@functools.lru_cache(maxsize=None)
def _k1():
    return pl.kernel(
        _k1_body,
        out_type=(
            jax.ShapeDtypeStruct((_H * _W // 128, 128), jnp.int32),
            jax.ShapeDtypeStruct((_H * _W,), jnp.float32),
            jax.ShapeDtypeStruct((_NW * 32,), jnp.float32),
        ),
        mesh=_mesh(),
        compiler_params=pltpu.CompilerParams(needs_layout_passes=False, use_tc_tiling_on_sc=False),
        scratch_types=[
            pltpu.VMEM((_RB,), jnp.int32),
            pltpu.VMEM((_W, _RB), jnp.int32),
            pltpu.VMEM((4, 128), jnp.int32),
            pltpu.VMEM((_RB * _W,), jnp.float32),
            pltpu.VMEM((32,), jnp.float32),
            pltpu.SemaphoreType.DMA,
            pltpu.SemaphoreType.DMA,
            pltpu.SemaphoreType.DMA,
        ],
    )


def _k2_body(fi_hbm, dv_hbm, idtT_hbm, ta_hbm, part_hbm, outT_hbm,
             fiB0, fiB1, dvB0, dvB1, idB0, idB1, abB0, abB1, xd0, xd1,
             posB0, posB1, pbuf, semi0, semi1, semg0, semg1, semo0, semo1):
    wid = _worker_id()
    r_start = wid * _CHUNK
    nrows = jnp.minimum(_CHUNK, _H - r_start)
    nblk = nrows // _RB
    iota16 = lax.iota(jnp.int32, 16)
    ones16 = jnp.ones((16,), jnp.float32)
    zero16 = jnp.zeros((16,), jnp.float32)

    pltpu.sync_copy(part_hbm, pbuf)
    mn = pbuf[pl.ds(0, 16)]
    mx = pbuf[pl.ds(16, 16)]
    for i in range(1, _NW):
        mn = jnp.minimum(mn, pbuf[pl.ds(32 * i, 16)])
        mx = jnp.maximum(mx, pbuf[pl.ds(32 * i + 16, 16)])
    gmin = jnp.min(mn)
    inv_v = ones16 / (jnp.full((16,), 1.0, jnp.float32) * (jnp.max(mx) - gmin))

    def inload(b, fiB, dvB, idB, semi):
        r0 = r_start + b * _RB
        pltpu.async_copy(fi_hbm.at[pl.ds(r0 // 4, 4)], fiB, semi)
        pltpu.async_copy(dv_hbm.at[pl.ds(r0 * _W, _RB * _W)], dvB, semi)
        pltpu.async_copy(idtT_hbm.at[:, pl.ds(r0, _RB)], idB, semi)

    def process(b, fiB, dvB, idB, abB, xd, posB, semi, semg, semo,
                nfiB, ndvB, nidB, nsemi):
        r0 = r_start + b * _RB
        pltpu.make_async_copy(fi_hbm.at[pl.ds(0, 4)], fiB, semi).wait()
        pltpu.make_async_copy(dv_hbm.at[pl.ds(0, _RB * _W)], dvB, semi).wait()
        pltpu.make_async_copy(idtT_hbm.at[:, pl.ds(0, _RB)], idB, semi).wait()
        descs = [
            pltpu.async_copy(ta_hbm.at[fiB.at[g]],
                             abB.at[pl.ds(128 * g, 128)], semg)
            for g in range(4)
        ]

        @pl.when(b + 1 < nblk)
        def _():
            inload(b + 1, nfiB, ndvB, nidB, nsemi)

        @pl.when(b >= 2)
        def _():
            pltpu.make_async_copy(xd, outT_hbm.at[:, pl.ds(0, _RB)],
                                  semo).wait()
            for w in range(_W):
                plsc.store_scatter(xd, [posB[w, :], iota16], zero16)

        @pl.when(b < 2)
        def _():
            for c in range(_D1):
                xd[c, :] = zero16

        for w in range(_W):
            d = dvB[pl.ds(16 * w, 16)]
            xd[_D1 + w, :] = (d - gmin) * inv_v
        for w in range(_W):
            pos = 22 * w + idB[w, :]
            plsc.store_scatter(xd, [pos, iota16], ones16)
            posB[w, :] = pos
        for dsc in descs:
            dsc.wait()
        for w in range(_W):
            base = 16 * w + iota16
            for j in range(8):
                v = plsc.load_gather(abB, [base, jnp.full((16,), j, jnp.int32)])
                xd[_D2 + 8 * w + j, :] = v
        pltpu.async_copy(xd, outT_hbm.at[:, pl.ds(r0, _RB)], semo)

    inload(0, fiB0, dvB0, idB0, semi0)

    def blk(b, _):
        @pl.when(b % 2 == 0)
        def _():
            process(b, fiB0, dvB0, idB0, abB0, xd0, posB0, semi0, semg0,
                    semo0, fiB1, dvB1, idB1, semi1)

        @pl.when(b % 2 == 1)
        def _():
            process(b, fiB1, dvB1, idB1, abB1, xd1, posB1, semi1, semg1,
                    semo1, fiB0, dvB0, idB0, semi0)

        return 0

    lax.fori_loop(0, nblk, blk, 0)
    pltpu.make_async_copy(xd0, outT_hbm.at[:, pl.ds(0, _RB)], semo0).wait()
    pltpu.make_async_copy(xd1, outT_hbm.at[:, pl.ds(0, _RB)], semo1).wait()


@functools.lru_cache(maxsize=None)
def _k2():
    return pl.kernel(
        _k2_body,
        out_type=jax.ShapeDtypeStruct((_DOUT, _H), jnp.float32),
        mesh=_mesh(),
        compiler_params=pltpu.CompilerParams(needs_layout_passes=False, use_tc_tiling_on_sc=False),
        scratch_types=[
            pltpu.VMEM((4, 128), jnp.int32),
            pltpu.VMEM((4, 128), jnp.int32),
            pltpu.VMEM((_RB * _W,), jnp.float32),
            pltpu.VMEM((_RB * _W,), jnp.float32),
            pltpu.VMEM((_W, _RB), jnp.int32),
            pltpu.VMEM((_W, _RB), jnp.int32),
            pltpu.VMEM((_RB * _W, 8), jnp.float32),
            pltpu.VMEM((_RB * _W, 8), jnp.float32),
            pltpu.VMEM((_DOUT, _RB), jnp.float32),
            pltpu.VMEM((_DOUT, _RB), jnp.float32),
            pltpu.VMEM((_W, 16), jnp.int32),
            pltpu.VMEM((_W, 16), jnp.int32),
            pltpu.VMEM((_NW * 32,), jnp.float32),
            pltpu.SemaphoreType.DMA,
            pltpu.SemaphoreType.DMA,
            pltpu.SemaphoreType.DMA,
            pltpu.SemaphoreType.DMA,
            pltpu.SemaphoreType.DMA,
            pltpu.SemaphoreType.DMA,
        ],
    )


def kernel(dist, angle, idx_t, index_t, index_h):
    ta3, td2 = _build_tables(dist, angle)
    ta = ta3.reshape(_V, 2 * _A)
    td = td2.reshape(_V)
    fi, dv, part = _k1()(index_h, index_t.T, td)
    out_t = _k2()(fi, dv, idx_t.T, ta, part)
    return out_t.T
